# baseline (speedup 1.0000x reference)
import functools

import jax
import jax.numpy as jnp
from jax import lax
from jax.experimental import pallas as pl
from jax.experimental.pallas import tpu as pltpu
from jax.experimental.pallas import tpu_sc as plsc

_INFO = plsc.get_sparse_core_info()
_NC = _INFO.num_cores
_NS = _INFO.num_subcores
_NW = _NC * _NS

_S = 50
_T = 16384
_D = 32
_TL = 128
_DS = 8
_NDT = _D // _DS
_NTT = _T // _TL
_NBLK = _S * _NTT
_BPW = _NBLK // _NW


def _make_gather():
    mesh = plsc.VectorSubcoreMesh(core_axis_name="c", subcore_axis_name="s")

    @functools.partial(
        pl.kernel,
        mesh=mesh,
        out_type=jax.ShapeDtypeStruct((_S, _NDT, _NTT, _DS * _TL),
                                      jnp.float32),
        scratch_types=[
            pltpu.VMEM((_BPW, _TL), jnp.int32),
            pltpu.VMEM((_TL, _D), jnp.float32),
            pltpu.VMEM((_TL, _D), jnp.float32),
            pltpu.VMEM((_D * _TL + _TL,), jnp.float32),
            pltpu.VMEM((_D * _TL + _TL,), jnp.float32),
            pltpu.SemaphoreType.DMA,
            pltpu.SemaphoreType.DMA,
            pltpu.SemaphoreType.DMA,
            pltpu.SemaphoreType.DMA,
        ],
        compiler_params=pltpu.CompilerParams(use_tc_tiling_on_sc=False,
                                             needs_layout_passes=False),
    )
    def emb(table_hbm, idx_hbm, out_hbm, idx_v, rows0, rows1, tb0, tb1,
            gsem0, gsem1, ssem0, ssem1):
        wid = lax.axis_index("s") * _NC + lax.axis_index("c")
        base = wid * _BPW
        pltpu.sync_copy(idx_hbm.at[wid], idx_v)

        rows = (rows0, rows1)
        tbs = (tb0, tb1)
        gsems = (gsem0, gsem1)
        ssems = (ssem0, ssem1)

        base_d = lax.iota(jnp.int32, 16) * _TL
        base_u = [base_d + u for u in range(8)]

        pltpu.async_copy(table_hbm.at[idx_v.at[0]], rows0, gsem0)
        pltpu.async_copy(table_hbm.at[idx_v.at[1]], rows1, gsem1)

        def step(k2, _):
            for b in range(2):
                k = k2 * 2 + b
                rbuf, tbuf, gsem, ssem = rows[b], tbs[b], gsems[b], ssems[b]
                pltpu.make_async_copy(table_hbm.at[idx_v.at[k]], rbuf,
                                      gsem).wait()

                @pl.when(k >= 2)
                def _():
                    bid2 = base + k - 2
                    s2 = bid2 // _NTT
                    tt2 = lax.rem(bid2, _NTT)
                    for dt in range(_NDT):
                        pltpu.make_async_copy(
                            tbuf.at[pl.ds(dt * _DS * _TL, _DS * _TL)],
                            out_hbm.at[s2, dt, tt2], ssem).wait()

                half = 16 * _TL

                def tr(tj, _):
                    t8 = tj * 8
                    lo_ref = tbuf.at[pl.ds(t8, half)]
                    hi_ref = tbuf.at[pl.ds(t8 + half, half)]
                    for u in range(8):
                        plsc.store_scatter(lo_ref, [base_u[u]],
                                           rbuf[t8 + u, pl.ds(0, 16)])
                        plsc.store_scatter(hi_ref, [base_u[u]],
                                           rbuf[t8 + u, pl.ds(16, 16)])
                    return 0

                lax.fori_loop(0, _TL // 8, tr, 0)

                bid = base + k
                s = bid // _NTT
                tt = lax.rem(bid, _NTT)
                for dt in range(_NDT):
                    pltpu.async_copy(
                        tbuf.at[pl.ds(dt * _DS * _TL, _DS * _TL)],
                        out_hbm.at[s, dt, tt], ssem)

                @pl.when(k + 2 < _BPW)
                def _():
                    pltpu.async_copy(table_hbm.at[idx_v.at[k + 2]], rbuf, gsem)

            return 0

        lax.fori_loop(0, _BPW // 2, step, 0)

        for b in range(2):
            k = _BPW - 2 + b
            bid = base + k
            s = bid // _NTT
            tt = lax.rem(bid, _NTT)
            for dt in range(_NDT):
                pltpu.make_async_copy(
                    tbs[b].at[pl.ds(dt * _DS * _TL, _DS * _TL)],
                    out_hbm.at[s, dt, tt], ssems[b]).wait()

    return emb


def kernel(token_ids, weight):
    idsw = token_ids.astype(jnp.int32).T.reshape(_NW, _BPW, _TL)
    out5 = _make_gather()(weight, idsw)
    out5 = out5.reshape(_S, _NDT, _NTT, _DS, _TL)
    return out5.transpose(2, 4, 0, 1, 3).reshape(_T, _S, _D)


# speedup vs baseline: 1.0012x; 1.0012x over previous
import functools

import jax
import jax.numpy as jnp
from jax import lax
from jax.experimental import pallas as pl
from jax.experimental.pallas import tpu as pltpu
from jax.experimental.pallas import tpu_sc as plsc

_INFO = plsc.get_sparse_core_info()
_NC = _INFO.num_cores
_NS = _INFO.num_subcores
_NW = _NC * _NS

_S = 50
_T = 16384
_D = 32
_TL = 128
_DS = 8
_NDT = _D // _DS
_TB = 512
_NQ = _T // _TB
_NBLK = _S * _NQ
_BPW = _NBLK // _NW

_DT_RUN = _DS * _TB
_TBUF = _D * _TB + 256


def _make_gather():
    mesh = plsc.VectorSubcoreMesh(core_axis_name="c", subcore_axis_name="s")

    @functools.partial(
        pl.kernel,
        mesh=mesh,
        out_type=jax.ShapeDtypeStruct((_S, _NDT, _T * _DS), jnp.float32),
        scratch_types=[
            pltpu.VMEM((_BPW, _TB), jnp.int32),
            pltpu.VMEM((_TB, _D), jnp.float32),
            pltpu.VMEM((_TB, _D), jnp.float32),
            pltpu.VMEM((_TBUF,), jnp.float32),
            pltpu.VMEM((_TBUF,), jnp.float32),
            pltpu.SemaphoreType.DMA,
            pltpu.SemaphoreType.DMA,
            pltpu.SemaphoreType.DMA,
            pltpu.SemaphoreType.DMA,
        ],
        compiler_params=pltpu.CompilerParams(use_tc_tiling_on_sc=False,
                                             needs_layout_passes=False),
    )
    def emb(table_hbm, idx_hbm, out_hbm, idx_v, rows0, rows1, tb0, tb1,
            gsem0, gsem1, ssem0, ssem1):
        wid = lax.axis_index("s") * _NC + lax.axis_index("c")
        base = wid * _BPW
        pltpu.sync_copy(idx_hbm.at[wid], idx_v)

        rows = (rows0, rows1)
        tbs = (tb0, tb1)
        gsems = (gsem0, gsem1)
        ssems = (ssem0, ssem1)

        lanes = lax.iota(jnp.int32, 16)
        base_dv = (lanes // 8) * (_NDT * 1024) + (lanes % 8) * _TL
        base_u = [base_dv + u for u in range(8)]
        _HI = 16 // 8 * (_NDT * 1024)
        _SPAN = 5120

        def start_gather(k, rbuf, gsem):
            for h in range(2):
                pltpu.async_copy(
                    table_hbm.at[idx_v.at[k, pl.ds(h * 256, 256)]],
                    rbuf.at[pl.ds(h * 256, 256)], gsem)

        def wait_gather(k, rbuf, gsem):
            for h in range(2):
                pltpu.make_async_copy(
                    table_hbm.at[idx_v.at[k, pl.ds(h * 256, 256)]],
                    rbuf.at[pl.ds(h * 256, 256)], gsem).wait()

        start_gather(0, rows0, gsem0)
        start_gather(1, rows1, gsem1)

        def stores(bid, tbuf, ssem, wait):
            s = bid // _NQ
            q = lax.rem(bid, _NQ)
            for dt in range(_NDT):
                src = tbuf.at[pl.ds(dt * _DT_RUN, _DT_RUN)]
                dst = out_hbm.at[s, dt, pl.ds(q * _DT_RUN, _DT_RUN)]
                if wait:
                    pltpu.make_async_copy(src, dst, ssem).wait()
                else:
                    pltpu.async_copy(src, dst, ssem)

        def step(k2, _):
            for b in range(2):
                k = k2 * 2 + b
                rbuf, tbuf, gsem, ssem = rows[b], tbs[b], gsems[b], ssems[b]
                wait_gather(k, rbuf, gsem)

                @pl.when(k >= 2)
                def _():
                    stores(base + k - 2, tbuf, ssem, wait=True)

                def tr(tj, _):
                    t8 = tj * 8
                    off = (tj // 16) * 1024 + lax.rem(tj, 16) * 8
                    lo_ref = tbuf.at[pl.ds(off, _SPAN)]
                    hi_ref = tbuf.at[pl.ds(off + _HI, _SPAN)]
                    rb = rbuf.at[pl.ds(t8, 8)]
                    for u in range(8):
                        plsc.store_scatter(lo_ref, [base_u[u]],
                                           rb[u, pl.ds(0, 16)])
                        plsc.store_scatter(hi_ref, [base_u[u]],
                                           rb[u, pl.ds(16, 16)])
                    return 0

                lax.fori_loop(0, _TB // 8, tr, 0)

                stores(base + k, tbuf, ssem, wait=False)

                @pl.when(k + 2 < _BPW)
                def _():
                    start_gather(k + 2, rbuf, gsem)

            return 0

        lax.fori_loop(0, _BPW // 2, step, 0)

        for b in range(2):
            stores(base + _BPW - 2 + b, tbs[b], ssems[b], wait=True)

    return emb


def kernel(token_ids, weight):
    idsw = token_ids.astype(jnp.int32).T.reshape(_NW, _BPW, _TB)
    out3 = _make_gather()(weight, idsw)
    out5 = out3.reshape(_S, _NDT, _T // _TL, _DS, _TL)
    return out5.transpose(2, 4, 0, 1, 3).reshape(_T, _S, _D)


# speedup vs baseline: 1.4073x; 1.4057x over previous
import functools

import jax
import jax.numpy as jnp
from jax import lax
from jax.experimental import pallas as pl
from jax.experimental.pallas import tpu as pltpu
from jax.experimental.pallas import tpu_sc as plsc

_INFO = plsc.get_sparse_core_info()
_NC = _INFO.num_cores
_NS = _INFO.num_subcores
_NW = _NC * _NS

_S = 50
_T = 16384
_D = 32
_TL = 128
_DS = 8
_NDT = _D // _DS
_TB = 512
_NQ = _T // _TB
_NBLK = _S * _NQ
_BPW = _NBLK // _NW

_DT_RUN = _DS * _TB
_TBUF = _D * _TB + 256


def _make_gather():
    mesh = plsc.VectorSubcoreMesh(core_axis_name="c", subcore_axis_name="s")

    @functools.partial(
        pl.kernel,
        mesh=mesh,
        out_type=jax.ShapeDtypeStruct((_S, _NDT, _T * _DS), jnp.float32),
        scratch_types=[
            pltpu.VMEM((_BPW, _TB), jnp.int32),
            pltpu.VMEM((_TB, _D), jnp.float32),
            pltpu.VMEM((_TB, _D), jnp.float32),
            pltpu.VMEM((_TBUF,), jnp.float32),
            pltpu.VMEM((_TBUF,), jnp.float32),
            pltpu.SemaphoreType.DMA,
            pltpu.SemaphoreType.DMA,
            pltpu.SemaphoreType.DMA,
            pltpu.SemaphoreType.DMA,
        ],
        compiler_params=pltpu.CompilerParams(use_tc_tiling_on_sc=False,
                                             needs_layout_passes=False),
    )
    def emb(table_hbm, idx_hbm, out_hbm, idx_v, rows0, rows1, tb0, tb1,
            gsem0, gsem1, ssem0, ssem1):
        wid = lax.axis_index("s") * _NC + lax.axis_index("c")
        base = wid * _BPW
        pltpu.sync_copy(idx_hbm.at[wid], idx_v)

        rows = (rows0, rows1)
        tbs = (tb0, tb1)
        gsems = (gsem0, gsem1)
        ssems = (ssem0, ssem1)

        lanes = lax.iota(jnp.int32, 16)
        col_k = [lax.rem(lanes + k, 16) for k in range(16)]
        col_hi_k = [c + 16 for c in col_k]
        dst_k = [(c // 8) * 4096 + lax.rem(c, 8) * _TL + lanes for c in col_k]
        _HI = 8192
        _SPAN = 5120

        def start_gather(k, rbuf, gsem):
            for h in range(2):
                pltpu.async_copy(
                    table_hbm.at[idx_v.at[k, pl.ds(h * 256, 256)]],
                    rbuf.at[pl.ds(h * 256, 256)], gsem)

        def wait_gather(k, rbuf, gsem):
            for h in range(2):
                pltpu.make_async_copy(
                    table_hbm.at[idx_v.at[k, pl.ds(h * 256, 256)]],
                    rbuf.at[pl.ds(h * 256, 256)], gsem).wait()

        start_gather(0, rows0, gsem0)
        start_gather(1, rows1, gsem1)

        def stores(bid, tbuf, ssem, wait):
            s = bid // _NQ
            q = lax.rem(bid, _NQ)
            for dt in range(_NDT):
                src = tbuf.at[pl.ds(dt * _DT_RUN, _DT_RUN)]
                dst = out_hbm.at[s, dt, pl.ds(q * _DT_RUN, _DT_RUN)]
                if wait:
                    pltpu.make_async_copy(src, dst, ssem).wait()
                else:
                    pltpu.async_copy(src, dst, ssem)

        def step(k2, _):
            for b in range(2):
                k = k2 * 2 + b
                rbuf, tbuf, gsem, ssem = rows[b], tbs[b], gsems[b], ssems[b]
                wait_gather(k, rbuf, gsem)

                @pl.when(k >= 2)
                def _():
                    stores(base + k - 2, tbuf, ssem, wait=True)

                def tr(tg, _):
                    t0 = tg * 16
                    off = (tg // 8) * 1024 + lax.rem(tg, 8) * 16
                    rv = lanes + t0
                    lo_ref = tbuf.at[pl.ds(off, _SPAN)]
                    hi_ref = tbuf.at[pl.ds(off + _HI, _SPAN)]
                    for k in range(16):
                        plsc.store_scatter(
                            lo_ref, [dst_k[k]],
                            plsc.load_gather(rbuf, [rv, col_k[k]]))
                        plsc.store_scatter(
                            hi_ref, [dst_k[k]],
                            plsc.load_gather(rbuf, [rv, col_hi_k[k]]))
                    return 0

                lax.fori_loop(0, _TB // 16, tr, 0)

                stores(base + k, tbuf, ssem, wait=False)

                @pl.when(k + 2 < _BPW)
                def _():
                    start_gather(k + 2, rbuf, gsem)

            return 0

        lax.fori_loop(0, _BPW // 2, step, 0)

        for b in range(2):
            stores(base + _BPW - 2 + b, tbs[b], ssems[b], wait=True)

    return emb


def kernel(token_ids, weight):
    idsw = token_ids.astype(jnp.int32).T.reshape(_NW, _BPW, _TB)
    out3 = _make_gather()(weight, idsw)
    out5 = out3.reshape(_S, _NDT, _T // _TL, _DS, _TL)
    return out5.transpose(2, 4, 0, 1, 3).reshape(_T, _S, _D)
